# baseline (speedup 1.0000x reference)
import functools

import jax
import jax.numpy as jnp
from jax import lax
from jax.experimental import pallas as pl
from jax.experimental.pallas import tpu as pltpu
from jax.experimental.pallas import tpu_sc as plsc

NUM_EMB = 1_000_000
D_IN = 16
D_OUT = 32
BATCH = 16384
FIELDS = 26
B = BATCH * FIELDS

NC = 2
NS = 16
NW = NC * NS

GROUP = 128
G_PER_W = B // (NW * GROUP)
CHUNK_G = 8
N_CHUNKS = G_PER_W // CHUNK_G
IDX_ROWS = B // GROUP

PACK = 128 // D_IN
TP_ROWS = NUM_EMB // PACK
FULL_TILES = NUM_EMB // 128
TAIL0 = FULL_TILES * 128
BASE_T = FULL_TILES // NW
EXTRA_T = FULL_TILES - BASE_T * NW

_sc_mesh = plsc.VectorSubcoreMesh(core_axis_name="c", subcore_axis_name="s")


TT_CW = 512
TT_OR = TT_CW // PACK
TT_CHUNKS = (NUM_EMB // 128) // (TT_CW // 128)
TT_BASE = TT_CHUNKS // NW
TT_EXTRA = TT_CHUNKS - TT_BASE * NW


@functools.partial(
    pl.kernel,
    out_type=jax.ShapeDtypeStruct((TP_ROWS, 128), jnp.float32),
    mesh=_sc_mesh,
    scratch_types=[
        pltpu.VMEM((2, D_IN, TT_CW + 1), jnp.float32),
        pltpu.VMEM((2, TT_OR, 128), jnp.float32),
        pltpu.SemaphoreType.DMA((2,)),
        pltpu.SemaphoreType.DMA((2,)),
    ],
    compiler_params=pltpu.CompilerParams(
        use_tc_tiling_on_sc=True, needs_layout_passes=False
    ),
)
def _sc_transpose(emb_t_hbm, tail_hbm, out_hbm, in_v, out_v, isem, osem):
    wid = lax.axis_index("s") * NC + lax.axis_index("c")
    nchunks = TT_BASE + jnp.where(wid < TT_EXTRA, 1, 0)
    c0 = TT_BASE * wid + jnp.minimum(wid, TT_EXTRA)
    row_iota = lax.iota(jnp.int32, 16)

    def start_in(c, slot):
        pltpu.async_copy(
            emb_t_hbm.at[:, pl.ds((c0 + c) * TT_CW, TT_CW)],
            in_v.at[slot, :, pl.ds(0, TT_CW)],
            isem.at[slot],
        )

    def wait_in(slot):
        pltpu.make_async_copy(
            emb_t_hbm.at[:, pl.ds(0, TT_CW)],
            in_v.at[slot, :, pl.ds(0, TT_CW)],
            isem.at[slot],
        ).wait()

    def start_out(c, slot):
        pltpu.async_copy(
            out_v.at[slot],
            out_hbm.at[pl.ds((c0 + c) * TT_OR, TT_OR)],
            osem.at[slot],
        )

    def wait_out(slot):
        pltpu.make_async_copy(
            out_v.at[slot], out_hbm.at[pl.ds(0, TT_OR)], osem.at[slot]
        ).wait()

    start_in(0, 0)

    def chunk(c, carry):
        slot = c % 2

        @pl.when(c + 1 < nchunks)
        def _():
            start_in(c + 1, (c + 1) % 2)

        wait_in(slot)

        @pl.when(c >= 2)
        def _():
            wait_out(slot)

        slot_full = jnp.full((16,), slot, jnp.int32)
        lane_s = [row_iota + s * D_IN for s in range(PACK)]
        one = jnp.full((16,), 1, jnp.int32)

        def cols(j0, cvec):
            for jj in range(0, 128, 32):
                cv, loads = cvec, []
                for t in range(32):
                    loads.append(
                        (plsc.load_gather(in_v, [slot_full, row_iota, cv]),
                         cv)
                    )
                    cv = cv + one
                for t, (col, cvt) in enumerate(loads):
                    plsc.store_scatter(
                        out_v,
                        [slot_full, cvt >> 3, lane_s[(jj + t) % PACK]],
                        col,
                    )
                cvec = cv
            return cvec

        zero16 = jnp.full((16,), 0, jnp.int32)
        lax.fori_loop(0, TT_CW // 128, cols, zero16)
        start_out(c, slot)
        return carry

    lax.fori_loop(0, nchunks, chunk, 0)
    wait_out(nchunks % 2)
    wait_out((nchunks + 1) % 2)

    @pl.when(wid == NW - 1)
    def _():
        pltpu.sync_copy(tail_hbm, out_v.at[0, pl.ds(0, PACK)])
        pltpu.sync_copy(out_v.at[0, pl.ds(0, PACK)],
                        out_hbm.at[pl.ds(TP_ROWS - PACK, PACK)])


@functools.partial(
    pl.kernel,
    out_type=jax.ShapeDtypeStruct((B, D_IN), jnp.float32),
    mesh=_sc_mesh,
    scratch_types=[
        pltpu.VMEM((G_PER_W, GROUP), jnp.int32),
        pltpu.VMEM((CHUNK_G * GROUP, D_IN), jnp.float32),
        pltpu.SemaphoreType.DMA,
    ],
    compiler_params=pltpu.CompilerParams(use_tc_tiling_on_sc=False),
)
def _sc_gather(idx_hbm, table_hbm, x_hbm, idx_v, rows_v, sem):
    wid = lax.axis_index("s") * NC + lax.axis_index("c")
    g0 = wid * G_PER_W
    pltpu.sync_copy(idx_hbm.at[pl.ds(g0, G_PER_W)], idx_v)

    def chunk(i, carry):
        base_g = i * CHUNK_G
        cps = [
            pltpu.async_copy(
                table_hbm.at[idx_v.at[base_g + j]],
                rows_v.at[pl.ds(j * GROUP, GROUP)],
                sem,
            )
            for j in range(CHUNK_G)
        ]
        for cp in cps:
            cp.wait()
        row0 = (g0 + base_g) * GROUP
        pltpu.sync_copy(rows_v, x_hbm.at[pl.ds(row0, CHUNK_G * GROUP)])
        return carry

    lax.fori_loop(0, N_CHUNKS, chunk, 0)


PBF = BATCH // PACK


def _tc_matmul(x_ref, f_ref, o_ref):
    for s in range(PACK):
        y = lax.dot_general(
            f_ref[s], x_ref[...],
            (((1,), (1,)), ((), ())),
            preferred_element_type=jnp.float32,
        )
        o_ref[0, :, pl.ds(s * PBF, PBF)] = y


def kernel(inputs, embedding, factor_tensor):
    tail8 = embedding[TAIL0:, :].reshape(PACK, 128)
    tpack = _sc_transpose(embedding.T, tail8)
    table_rows = tpack.reshape(NUM_EMB, D_IN)

    idx2d = (inputs.T.reshape(FIELDS, PACK, BATCH // PACK)
             .transpose(0, 2, 1).reshape(IDX_ROWS, GROUP))
    x = _sc_gather(idx2d, table_rows)
    x2 = x.reshape(B // PACK, PACK * D_IN)

    f_exp = jnp.zeros((PACK, D_OUT, PACK * D_IN), jnp.float32)
    for s in range(PACK):
        f_exp = f_exp.at[s, :, s * D_IN:(s + 1) * D_IN].set(factor_tensor.T)

    out = pl.pallas_call(
        _tc_matmul,
        grid=(FIELDS,),
        in_specs=[
            pl.BlockSpec((PBF, PACK * D_IN), lambda i: (i, 0)),
            pl.BlockSpec((PACK, D_OUT, PACK * D_IN), lambda i: (0, 0, 0)),
        ],
        out_specs=pl.BlockSpec((1, D_OUT, BATCH), lambda i: (i, 0, 0)),
        out_shape=jax.ShapeDtypeStruct((FIELDS, D_OUT, BATCH), jnp.float32),
    )(x2, f_exp)
    return out.transpose(2, 0, 1)


# speedup vs baseline: 1.0093x; 1.0093x over previous
import functools

import jax
import jax.numpy as jnp
from jax import lax
from jax.experimental import pallas as pl
from jax.experimental.pallas import tpu as pltpu
from jax.experimental.pallas import tpu_sc as plsc

NUM_EMB = 1_000_000
D_IN = 16
D_OUT = 32
BATCH = 16384
FIELDS = 26
B = BATCH * FIELDS

NC = 2
NS = 16
NW = NC * NS

GROUP = 128
G_PER_W = B // (NW * GROUP)
CHUNK_G = 8
N_CHUNKS = G_PER_W // CHUNK_G
IDX_ROWS = B // GROUP

PACK = 128 // D_IN
TP_ROWS = NUM_EMB // PACK
FULL_TILES = NUM_EMB // 128
TAIL0 = FULL_TILES * 128
BASE_T = FULL_TILES // NW
EXTRA_T = FULL_TILES - BASE_T * NW

_sc_mesh = plsc.VectorSubcoreMesh(core_axis_name="c", subcore_axis_name="s")


TT_CW = 768
TT_OR = TT_CW // PACK
TT_CHUNKS = (NUM_EMB // 128) // (TT_CW // 128)
TT_BASE = TT_CHUNKS // NW
TT_EXTRA = TT_CHUNKS - TT_BASE * NW


@functools.partial(
    pl.kernel,
    out_type=jax.ShapeDtypeStruct((TP_ROWS, 128), jnp.float32),
    mesh=_sc_mesh,
    scratch_types=[
        pltpu.VMEM((2, D_IN, TT_CW + 1), jnp.float32),
        pltpu.VMEM((2, TT_OR, 128), jnp.float32),
        pltpu.SemaphoreType.DMA((2,)),
        pltpu.SemaphoreType.DMA((2,)),
    ],
    compiler_params=pltpu.CompilerParams(
        use_tc_tiling_on_sc=True, needs_layout_passes=False
    ),
)
def _sc_transpose(emb_t_hbm, tail_hbm, out_hbm, in_v, out_v, isem, osem):
    wid = lax.axis_index("s") * NC + lax.axis_index("c")
    nchunks = TT_BASE + jnp.where(wid < TT_EXTRA, 1, 0)
    c0 = TT_BASE * wid + jnp.minimum(wid, TT_EXTRA)
    row_iota = lax.iota(jnp.int32, 16)

    def start_in(c, slot):
        pltpu.async_copy(
            emb_t_hbm.at[:, pl.ds((c0 + c) * TT_CW, TT_CW)],
            in_v.at[slot, :, pl.ds(0, TT_CW)],
            isem.at[slot],
        )

    def wait_in(slot):
        pltpu.make_async_copy(
            emb_t_hbm.at[:, pl.ds(0, TT_CW)],
            in_v.at[slot, :, pl.ds(0, TT_CW)],
            isem.at[slot],
        ).wait()

    def start_out(c, slot):
        pltpu.async_copy(
            out_v.at[slot],
            out_hbm.at[pl.ds((c0 + c) * TT_OR, TT_OR)],
            osem.at[slot],
        )

    def wait_out(slot):
        pltpu.make_async_copy(
            out_v.at[slot], out_hbm.at[pl.ds(0, TT_OR)], osem.at[slot]
        ).wait()

    start_in(0, 0)

    def chunk(c, carry):
        slot = c % 2

        @pl.when(c + 1 < nchunks)
        def _():
            start_in(c + 1, (c + 1) % 2)

        wait_in(slot)

        @pl.when(c >= 2)
        def _():
            wait_out(slot)

        slot_full = jnp.full((16,), slot, jnp.int32)
        lane_s = [row_iota + s * D_IN for s in range(PACK)]
        one = jnp.full((16,), 1, jnp.int32)

        def cols(j0, cvec):
            for jj in range(0, 128, 16):
                cv, loads = cvec, []
                for t in range(16):
                    loads.append(
                        (plsc.load_gather(in_v, [slot_full, row_iota, cv]),
                         cv)
                    )
                    cv = cv + one
                for t, (col, cvt) in enumerate(loads):
                    plsc.store_scatter(
                        out_v,
                        [slot_full, cvt >> 3, lane_s[(jj + t) % PACK]],
                        col,
                    )
                cvec = cv
            return cvec

        zero16 = jnp.full((16,), 0, jnp.int32)
        lax.fori_loop(0, TT_CW // 128, cols, zero16)
        start_out(c, slot)
        return carry

    lax.fori_loop(0, nchunks, chunk, 0)
    wait_out(nchunks % 2)
    wait_out((nchunks + 1) % 2)

    @pl.when(wid == NW - 1)
    def _():
        pltpu.sync_copy(tail_hbm, out_v.at[0, pl.ds(0, PACK)])
        pltpu.sync_copy(out_v.at[0, pl.ds(0, PACK)],
                        out_hbm.at[pl.ds(TP_ROWS - PACK, PACK)])


@functools.partial(
    pl.kernel,
    out_type=jax.ShapeDtypeStruct((B, D_IN), jnp.float32),
    mesh=_sc_mesh,
    scratch_types=[
        pltpu.VMEM((G_PER_W, GROUP), jnp.int32),
        pltpu.VMEM((CHUNK_G * GROUP, D_IN), jnp.float32),
        pltpu.SemaphoreType.DMA,
    ],
    compiler_params=pltpu.CompilerParams(use_tc_tiling_on_sc=False),
)
def _sc_gather(idx_hbm, table_hbm, x_hbm, idx_v, rows_v, sem):
    wid = lax.axis_index("s") * NC + lax.axis_index("c")
    g0 = wid * G_PER_W
    pltpu.sync_copy(idx_hbm.at[pl.ds(g0, G_PER_W)], idx_v)

    def chunk(i, carry):
        base_g = i * CHUNK_G
        cps = [
            pltpu.async_copy(
                table_hbm.at[idx_v.at[base_g + j]],
                rows_v.at[pl.ds(j * GROUP, GROUP)],
                sem,
            )
            for j in range(CHUNK_G)
        ]
        for cp in cps:
            cp.wait()
        row0 = (g0 + base_g) * GROUP
        pltpu.sync_copy(rows_v, x_hbm.at[pl.ds(row0, CHUNK_G * GROUP)])
        return carry

    lax.fori_loop(0, N_CHUNKS, chunk, 0)


PBF = BATCH // PACK


def _tc_matmul(x_ref, f_ref, o_ref):
    for s in range(PACK):
        y = lax.dot_general(
            f_ref[s], x_ref[...],
            (((1,), (1,)), ((), ())),
            preferred_element_type=jnp.float32,
        )
        o_ref[0, :, pl.ds(s * PBF, PBF)] = y


def kernel(inputs, embedding, factor_tensor):
    tail8 = embedding[TAIL0:, :].reshape(PACK, 128)
    tpack = _sc_transpose(embedding.T, tail8)
    table_rows = tpack.reshape(NUM_EMB, D_IN)

    idx2d = (inputs.T.reshape(FIELDS, PACK, BATCH // PACK)
             .transpose(0, 2, 1).reshape(IDX_ROWS, GROUP))
    x = _sc_gather(idx2d, table_rows)
    x2 = x.reshape(B // PACK, PACK * D_IN)

    f_exp = jnp.zeros((PACK, D_OUT, PACK * D_IN), jnp.float32)
    for s in range(PACK):
        f_exp = f_exp.at[s, :, s * D_IN:(s + 1) * D_IN].set(factor_tensor.T)

    out = pl.pallas_call(
        _tc_matmul,
        grid=(FIELDS,),
        in_specs=[
            pl.BlockSpec((PBF, PACK * D_IN), lambda i: (i, 0)),
            pl.BlockSpec((PACK, D_OUT, PACK * D_IN), lambda i: (0, 0, 0)),
        ],
        out_specs=pl.BlockSpec((1, D_OUT, BATCH), lambda i: (i, 0, 0)),
        out_shape=jax.ShapeDtypeStruct((FIELDS, D_OUT, BATCH), jnp.float32),
    )(x2, f_exp)
    return out.transpose(2, 0, 1)


# speedup vs baseline: 1.0107x; 1.0014x over previous
import functools

import jax
import jax.numpy as jnp
from jax import lax
from jax.experimental import pallas as pl
from jax.experimental.pallas import tpu as pltpu
from jax.experimental.pallas import tpu_sc as plsc

NUM_EMB = 1_000_000
D_IN = 16
D_OUT = 32
BATCH = 16384
FIELDS = 26
B = BATCH * FIELDS

NC = 2
NS = 16
NW = NC * NS

GROUP = 128
G_PER_W = B // (NW * GROUP)
CHUNK_G = 8
N_CHUNKS = G_PER_W // CHUNK_G
IDX_ROWS = B // GROUP

PACK = 128 // D_IN
TP_ROWS = NUM_EMB // PACK
TAIL0 = (NUM_EMB // 128) * 128

_sc_mesh = plsc.VectorSubcoreMesh(core_axis_name="c", subcore_axis_name="s")


TT_CW = 768
TT_OR = TT_CW // PACK
TT_CHUNKS = (NUM_EMB // 128) // (TT_CW // 128)
TT_BASE = TT_CHUNKS // NW
TT_EXTRA = TT_CHUNKS - TT_BASE * NW


@functools.partial(
    pl.kernel,
    out_type=jax.ShapeDtypeStruct((TP_ROWS, 128), jnp.float32),
    mesh=_sc_mesh,
    scratch_types=[
        pltpu.VMEM((2, D_IN, TT_CW + 1), jnp.float32),
        pltpu.VMEM((2, TT_OR, 128), jnp.float32),
        pltpu.SemaphoreType.DMA((2,)),
        pltpu.SemaphoreType.DMA((2,)),
    ],
    compiler_params=pltpu.CompilerParams(
        use_tc_tiling_on_sc=True, needs_layout_passes=False
    ),
)
def _sc_transpose(emb_t_hbm, tail_hbm, out_hbm, in_v, out_v, isem, osem):
    wid = lax.axis_index("s") * NC + lax.axis_index("c")
    nchunks = TT_BASE + jnp.where(wid < TT_EXTRA, 1, 0)
    c0 = TT_BASE * wid + jnp.minimum(wid, TT_EXTRA)
    row_iota = lax.iota(jnp.int32, 16)

    def start_in(c, slot):
        pltpu.async_copy(
            emb_t_hbm.at[:, pl.ds((c0 + c) * TT_CW, TT_CW)],
            in_v.at[slot, :, pl.ds(0, TT_CW)],
            isem.at[slot],
        )

    def wait_in(slot):
        pltpu.make_async_copy(
            emb_t_hbm.at[:, pl.ds(0, TT_CW)],
            in_v.at[slot, :, pl.ds(0, TT_CW)],
            isem.at[slot],
        ).wait()

    def start_out(c, slot):
        pltpu.async_copy(
            out_v.at[slot],
            out_hbm.at[pl.ds((c0 + c) * TT_OR, TT_OR)],
            osem.at[slot],
        )

    def wait_out(slot):
        pltpu.make_async_copy(
            out_v.at[slot], out_hbm.at[pl.ds(0, TT_OR)], osem.at[slot]
        ).wait()

    start_in(0, 0)

    def chunk(c, carry):
        slot = c % 2

        @pl.when(c + 1 < nchunks)
        def _():
            start_in(c + 1, (c + 1) % 2)

        wait_in(slot)

        @pl.when(c >= 2)
        def _():
            wait_out(slot)

        slot_full = jnp.full((16,), slot, jnp.int32)
        lane_s = [row_iota + s * D_IN for s in range(PACK)]
        one = jnp.full((16,), 1, jnp.int32)

        def cols(j0, cvec):
            for jj in range(0, 128, 16):
                cv, loads = cvec, []
                for t in range(16):
                    loads.append(
                        (plsc.load_gather(in_v, [slot_full, row_iota, cv]),
                         cv)
                    )
                    cv = cv + one
                for t, (col, cvt) in enumerate(loads):
                    plsc.store_scatter(
                        out_v,
                        [slot_full, cvt >> 3, lane_s[(jj + t) % PACK]],
                        col,
                    )
                cvec = cv
            return cvec

        zero16 = jnp.full((16,), 0, jnp.int32)
        lax.fori_loop(0, TT_CW // 128, cols, zero16)
        start_out(c, slot)
        return carry

    lax.fori_loop(0, nchunks, chunk, 0)
    wait_out(nchunks % 2)
    wait_out((nchunks + 1) % 2)

    @pl.when(wid == NW - 1)
    def _():
        pltpu.sync_copy(tail_hbm, out_v.at[0, pl.ds(0, PACK)])
        pltpu.sync_copy(out_v.at[0, pl.ds(0, PACK)],
                        out_hbm.at[pl.ds(TP_ROWS - PACK, PACK)])


@functools.partial(
    pl.kernel,
    out_type=jax.ShapeDtypeStruct((B, D_IN), jnp.float32),
    mesh=_sc_mesh,
    scratch_types=[
        pltpu.VMEM((G_PER_W, GROUP), jnp.int32),
        pltpu.VMEM((CHUNK_G * GROUP, D_IN), jnp.float32),
        pltpu.SemaphoreType.DMA,
    ],
    compiler_params=pltpu.CompilerParams(use_tc_tiling_on_sc=False),
)
def _sc_gather(idx_hbm, table_hbm, x_hbm, idx_v, rows_v, sem):
    wid = lax.axis_index("s") * NC + lax.axis_index("c")
    g0 = wid * G_PER_W
    pltpu.sync_copy(idx_hbm.at[pl.ds(g0, G_PER_W)], idx_v)

    def chunk(i, carry):
        base_g = i * CHUNK_G
        cps = [
            pltpu.async_copy(
                table_hbm.at[idx_v.at[base_g + j]],
                rows_v.at[pl.ds(j * GROUP, GROUP)],
                sem,
            )
            for j in range(CHUNK_G)
        ]
        for cp in cps:
            cp.wait()
        row0 = (g0 + base_g) * GROUP
        pltpu.sync_copy(rows_v, x_hbm.at[pl.ds(row0, CHUNK_G * GROUP)])
        return carry

    lax.fori_loop(0, N_CHUNKS, chunk, 0)


PBF = BATCH // PACK


def _tc_matmul(x_ref, f_ref, o_ref):
    for s in range(PACK):
        y = lax.dot_general(
            f_ref[s], x_ref[...],
            (((1,), (1,)), ((), ())),
            preferred_element_type=jnp.float32,
        )
        o_ref[0, :, pl.ds(s * PBF, PBF)] = y


def kernel(inputs, embedding, factor_tensor):
    tail8 = embedding[TAIL0:, :].reshape(PACK, 128)
    tpack = _sc_transpose(embedding.T, tail8)
    table_rows = tpack.reshape(NUM_EMB, D_IN)

    idx2d = (inputs.T.reshape(FIELDS, PACK, BATCH // PACK)
             .transpose(0, 2, 1).reshape(IDX_ROWS, GROUP))
    x = _sc_gather(idx2d, table_rows)
    x2 = x.reshape(B // PACK, PACK * D_IN)

    f_exp = jnp.zeros((PACK, D_OUT, PACK * D_IN), jnp.float32)
    for s in range(PACK):
        f_exp = f_exp.at[s, :, s * D_IN:(s + 1) * D_IN].set(factor_tensor.T)

    out = pl.pallas_call(
        _tc_matmul,
        grid=(FIELDS,),
        in_specs=[
            pl.BlockSpec((PBF, PACK * D_IN), lambda i: (i, 0)),
            pl.BlockSpec((PACK, D_OUT, PACK * D_IN), lambda i: (0, 0, 0)),
        ],
        out_specs=pl.BlockSpec((1, D_OUT, BATCH), lambda i: (i, 0, 0)),
        out_shape=jax.ShapeDtypeStruct((FIELDS, D_OUT, BATCH), jnp.float32),
    )(x2, f_exp)
    return out.transpose(2, 0, 1)
